# Initial kernel scaffold; baseline (speedup 1.0000x reference)
import jax
import jax.numpy as jnp
from jax.experimental import pallas as pl


def kernel(node_attrs, edge_indices, edge_attrs, node_indices, edge_batch_indices, instruction_batch, distribution, node_prop_similarities, relation_similarity, weight_node_properties, weight_edge, weight_node_score, weight_relation_score):
    raise NotImplementedError("write your pallas kernel here")


# speedup vs baseline: 8.9091x; 8.9091x over previous
import functools

import jax
import jax.numpy as jnp
from jax import lax
from jax.experimental import pallas as pl
from jax.experimental.pallas import tpu as pltpu
from jax.experimental.pallas import tpu_sc as plsc

_N = 10000
_E = 320000
_H = 128
_P = 8
_B = 32

_TE = 16000
_TN = 2000
_NW = 32
_EW = _E // _NW
_L = 16
_NPAD = 10240
_RD = _NPAD // 128


def _edge_body(ea_ref, ebi_ref, inst_ref, we_ref, wrs_ref, out_ref):
    ebi = ebi_ref[...]
    iota_b = lax.broadcasted_iota(jnp.int32, (1, _B), 1)
    oh = (ebi == iota_b).astype(jnp.float32)
    instg = jnp.dot(oh, inst_ref[...], preferred_element_type=jnp.float32)
    y = jnp.dot(instg * ea_ref[...], we_ref[...],
                preferred_element_type=jnp.float32)
    y = jnp.where(y > 0, y, jnp.exp(y) - 1.0)
    out_ref[...] = jnp.sum(y * wrs_ref[...], axis=1, keepdims=True)


def _node_body(na_ref, seg_ref, inst_ref, psT_ref, wnp_ref, wns_ref, out_ref):
    seg = seg_ref[...]
    iota_b = lax.broadcasted_iota(jnp.int32, (1, _B), 1)
    oh = (seg == iota_b).astype(jnp.float32)
    acc = jnp.zeros((_TN, _H), jnp.float32)
    for p in range(_P):
        ohp = oh * psT_ref[p:p + 1, :]
        g = jnp.dot(ohp, inst_ref[...], preferred_element_type=jnp.float32)
        acc = acc + jnp.dot(g * na_ref[:, p, :], wnp_ref[p],
                            preferred_element_type=jnp.float32)
    y = jnp.where(acc > 0, acc, jnp.exp(acc) - 1.0)
    out_ref[...] = jnp.sum(y * wns_ref[...], axis=1, keepdims=True)


def _combine_body(parts_ref, xs_ref, seg_ref, rs_ref, out_ref):
    xr = jnp.sum(parts_ref[...], axis=0)
    xs = xs_ref[...]
    seg = seg_ref[...]
    neg = jnp.float32(-1e30)
    mr_sel = jnp.zeros_like(xr)
    ms_sel = jnp.zeros_like(xs)
    for b in range(_B):
        m = seg == b
        mr = jnp.max(jnp.where(m, xr, neg))
        ms = jnp.max(jnp.where(m, xs, neg))
        mr = jnp.where(mr > jnp.float32(-5e29), mr, 0.0)
        ms = jnp.where(ms > jnp.float32(-5e29), ms, 0.0)
        mr_sel = mr_sel + jnp.where(m, mr, 0.0)
        ms_sel = ms_sel + jnp.where(m, ms, 0.0)
    er = jnp.exp(xr - mr_sel)
    es = jnp.exp(xs - ms_sel)
    sr_sel = jnp.ones_like(xr)
    ss_sel = jnp.ones_like(xs)
    rs_sel = jnp.zeros_like(xr)
    for b in range(_B):
        m = seg == b
        sr = jnp.sum(jnp.where(m, er, 0.0))
        ss = jnp.sum(jnp.where(m, es, 0.0))
        sr_sel = jnp.where(m, sr, sr_sel)
        ss_sel = jnp.where(m, ss, ss_sel)
        rs_sel = rs_sel + jnp.where(m, rs_ref[0, b], 0.0)
    out_ref[...] = rs_sel * (er / sr_sel) + (1.0 - rs_sel) * (es / ss_sel)


def _sc_scatter_body(src_hbm, dst_hbm, s_hbm, dist_hbm, out_hbm,
                     src_v, dst_v, s_v, dist_v, acc_v, kbuf, cbuf):
    wid = lax.axis_index("s") * 2 + lax.axis_index("c")
    base = wid * _EW
    pltpu.sync_copy(src_hbm.at[pl.ds(base, _EW)], src_v)
    pltpu.sync_copy(dst_hbm.at[pl.ds(base, _EW)], dst_v)
    pltpu.sync_copy(s_hbm.at[pl.ds(base, _EW)], s_v)
    pltpu.sync_copy(dist_hbm, dist_v)

    def zero_body(i, carry):
        acc_v[pl.ds(i * _L, _L)] = jnp.zeros((_L,), jnp.float32)
        return carry

    lax.fori_loop(0, _NPAD // _L, zero_body, 0)

    lane = lax.iota(jnp.int32, _L)

    def edge_body(i, carry):
        b = i * _L
        sv = s_v[pl.ds(b, _L)]
        srcv = src_v[pl.ds(b, _L)]
        dstv = dst_v[pl.ds(b, _L)]
        dv = plsc.load_gather(dist_v, [srcv])
        val = sv * dv
        k, v = plsc.sort_key_val(dstv, val)
        c = plsc.cumsum(v)
        kbuf[pl.ds(8, _L)] = k
        cbuf[...] = c
        kprev = kbuf[pl.ds(7, _L)]
        knext = kbuf[pl.ds(9, _L)]
        first = k != kprev
        last = (k != knext) | (lane == _L - 1)
        start = plsc.cummax(jnp.where(first, lane, 0))
        g = plsc.load_gather(cbuf, [jnp.maximum(start - 1, 0)])
        run_sum = c - jnp.where(start == 0, jnp.float32(0.0), g)
        plsc.addupdate_scatter(acc_v, [k], run_sum, mask=last)
        return carry

    lax.fori_loop(0, _EW // _L, edge_body, 0)
    pltpu.sync_copy(acc_v, out_hbm.at[wid])


@functools.lru_cache(maxsize=1)
def _make_sc_scatter():
    return pl.kernel(
        _sc_scatter_body,
        out_type=jax.ShapeDtypeStruct((_NW, _NPAD), jnp.float32),
        mesh=plsc.VectorSubcoreMesh(core_axis_name="c", subcore_axis_name="s"),
        compiler_params=pltpu.CompilerParams(needs_layout_passes=False),
        scratch_types=[
            pltpu.VMEM((_EW,), jnp.int32),
            pltpu.VMEM((_EW,), jnp.int32),
            pltpu.VMEM((_EW,), jnp.float32),
            pltpu.VMEM((_NPAD,), jnp.float32),
            pltpu.VMEM((_NPAD,), jnp.float32),
            pltpu.VMEM((32,), jnp.int32),
            pltpu.VMEM((_L,), jnp.float32),
        ],
    )


def kernel(node_attrs, edge_indices, edge_attrs, node_indices, edge_batch_indices,
           instruction_batch, distribution, node_prop_similarities, relation_similarity,
           weight_node_properties, weight_edge, weight_node_score, weight_relation_score):
    s2 = pl.pallas_call(
        _edge_body,
        grid=(_E // _TE,),
        in_specs=[
            pl.BlockSpec((_TE, _H), lambda i: (i, 0)),
            pl.BlockSpec((_TE, 1), lambda i: (i, 0)),
            pl.BlockSpec((_B, _H), lambda i: (0, 0)),
            pl.BlockSpec((_H, _H), lambda i: (0, 0)),
            pl.BlockSpec((1, _H), lambda i: (0, 0)),
        ],
        out_specs=pl.BlockSpec((_TE, 1), lambda i: (i, 0)),
        out_shape=jax.ShapeDtypeStruct((_E, 1), jnp.float32),
    )(edge_attrs, edge_batch_indices.reshape(_E, 1), instruction_batch,
      weight_edge, weight_relation_score.reshape(1, _H))

    dist_pad = jnp.pad(distribution, (0, _NPAD - _N))
    parts = _make_sc_scatter()(edge_indices[0], edge_indices[1],
                               s2.reshape(_E), dist_pad)

    xs2 = pl.pallas_call(
        _node_body,
        grid=(_N // _TN,),
        in_specs=[
            pl.BlockSpec((_TN, _P, _H), lambda i: (i, 0, 0)),
            pl.BlockSpec((_TN, 1), lambda i: (i, 0)),
            pl.BlockSpec((_B, _H), lambda i: (0, 0)),
            pl.BlockSpec((_P, _B), lambda i: (0, 0)),
            pl.BlockSpec((_P, _H, _H), lambda i: (0, 0, 0)),
            pl.BlockSpec((1, _H), lambda i: (0, 0)),
        ],
        out_specs=pl.BlockSpec((_TN, 1), lambda i: (i, 0)),
        out_shape=jax.ShapeDtypeStruct((_N, 1), jnp.float32),
    )(node_attrs, node_indices.reshape(_N, 1), instruction_batch,
      node_prop_similarities.T, weight_node_properties,
      weight_node_score.reshape(1, _H))

    xs_pad = jnp.pad(xs2.reshape(_N), (0, _NPAD - _N)).reshape(_RD, 128)
    seg_pad = jnp.pad(node_indices, (0, _NPAD - _N),
                      constant_values=-1).reshape(_RD, 128)
    out2 = pl.pallas_call(
        _combine_body,
        out_shape=jax.ShapeDtypeStruct((_RD, 128), jnp.float32),
    )(parts.reshape(_NW, _RD, 128), xs_pad, seg_pad,
      relation_similarity.reshape(1, _B))
    return out2.reshape(_NPAD)[:_N]


# speedup vs baseline: 22.1281x; 2.4838x over previous
import functools

import jax
import jax.numpy as jnp
from jax import lax
from jax.experimental import pallas as pl
from jax.experimental.pallas import tpu as pltpu
from jax.experimental.pallas import tpu_sc as plsc

_N = 10000
_E = 320000
_H = 128
_P = 8
_B = 32

_TE = 16384
_ER = _E // 128
_GE = -(-_ER // (_TE // 128))
_TN = 2048
_NW = 32
_EW = _E // _NW
_L = 16
_NPAD = 10240
_RD = _NPAD // 128
_GN = _NPAD // _TN


def _edge_body(ebi_ref, ea_ref, inst_ref, we_ref, wrs_ref, out_ref):
    ebi_row = ebi_ref[...].reshape(1, _TE)
    iota_b = lax.broadcasted_iota(jnp.int32, (_B, 1), 0)
    ohT = (ebi_row == iota_b).astype(jnp.float32)
    instg = lax.dot_general(ohT, inst_ref[...], (((0,), (0,)), ((), ())),
                            preferred_element_type=jnp.float32)
    y = jnp.dot(instg * ea_ref[...], we_ref[...],
                preferred_element_type=jnp.float32)
    y = jnp.where(y > 0, y, jnp.exp(y) - 1.0)
    s = jnp.dot(y, wrs_ref[...], preferred_element_type=jnp.float32)
    out_ref[...] = s.reshape(_TE // 128, 128)


def _node_body(seg_ref, na_ref, inst_ref, ps_ref, wnp_ref, wns_ref, out_ref):
    seg_row = seg_ref[...].reshape(1, _TN)
    iota_b = lax.broadcasted_iota(jnp.int32, (_B, 1), 0)
    ohT = (seg_row == iota_b).astype(jnp.float32)
    acc = jnp.zeros((_TN, _H), jnp.float32)
    for p in range(_P):
        ohpT = ohT * ps_ref[:, p:p + 1]
        g = lax.dot_general(ohpT, inst_ref[...], (((0,), (0,)), ((), ())),
                            preferred_element_type=jnp.float32)
        acc = acc + jnp.dot(g * na_ref[:, p, :], wnp_ref[p],
                            preferred_element_type=jnp.float32)
    y = jnp.where(acc > 0, acc, jnp.exp(acc) - 1.0)
    s = jnp.dot(y, wns_ref[...], preferred_element_type=jnp.float32)
    out_ref[...] = s.reshape(_TN // 128, 128)


def _combine_body(parts_ref, xs_ref, seg_ref, rs_ref, out_ref):
    xr = jnp.sum(parts_ref[...], axis=0)
    xs = xs_ref[...]
    seg = seg_ref[...]
    neg = jnp.float32(-1e30)
    mr_sel = jnp.zeros_like(xr)
    ms_sel = jnp.zeros_like(xs)
    for b in range(_B):
        m = seg == b
        mr = jnp.max(jnp.where(m, xr, neg))
        ms = jnp.max(jnp.where(m, xs, neg))
        mr = jnp.where(mr > jnp.float32(-5e29), mr, 0.0)
        ms = jnp.where(ms > jnp.float32(-5e29), ms, 0.0)
        mr_sel = mr_sel + jnp.where(m, mr, 0.0)
        ms_sel = ms_sel + jnp.where(m, ms, 0.0)
    er = jnp.exp(xr - mr_sel)
    es = jnp.exp(xs - ms_sel)
    sr_sel = jnp.ones_like(xr)
    ss_sel = jnp.ones_like(xs)
    rs_sel = jnp.zeros_like(xr)
    for b in range(_B):
        m = seg == b
        sr = jnp.sum(jnp.where(m, er, 0.0))
        ss = jnp.sum(jnp.where(m, es, 0.0))
        sr_sel = jnp.where(m, sr, sr_sel)
        ss_sel = jnp.where(m, ss, ss_sel)
        rs_sel = rs_sel + jnp.where(m, rs_ref[0, b], 0.0)
    out_ref[...] = rs_sel * (er / sr_sel) + (1.0 - rs_sel) * (es / ss_sel)


def _sc_scatter_body(src_hbm, dst_hbm, s_hbm, dist_hbm, out_hbm,
                     src_v, dst_v, s_v, dist_v, acc_v, kbuf, cbuf):
    wid = lax.axis_index("s") * 2 + lax.axis_index("c")
    base = wid * _EW
    pltpu.sync_copy(src_hbm.at[pl.ds(base, _EW)], src_v)
    pltpu.sync_copy(dst_hbm.at[pl.ds(base, _EW)], dst_v)
    pltpu.sync_copy(s_hbm.at[pl.ds(base, _EW)], s_v)
    pltpu.sync_copy(dist_hbm, dist_v)

    def zero_body(i, carry):
        acc_v[pl.ds(i * _L, _L)] = jnp.zeros((_L,), jnp.float32)
        return carry

    lax.fori_loop(0, _NPAD // _L, zero_body, 0)

    lane = lax.iota(jnp.int32, _L)

    def edge_body(i, carry):
        b = i * _L
        sv = s_v[pl.ds(b, _L)]
        srcv = src_v[pl.ds(b, _L)]
        dstv = dst_v[pl.ds(b, _L)]
        dv = plsc.load_gather(dist_v, [srcv])
        val = sv * dv
        k, v = plsc.sort_key_val(dstv, val)
        c = plsc.cumsum(v)
        kbuf[pl.ds(8, _L)] = k
        cbuf[...] = c
        kprev = kbuf[pl.ds(7, _L)]
        knext = kbuf[pl.ds(9, _L)]
        first = k != kprev
        last = (k != knext) | (lane == _L - 1)
        start = plsc.cummax(jnp.where(first, lane, 0))
        g = plsc.load_gather(cbuf, [jnp.maximum(start - 1, 0)])
        run_sum = c - jnp.where(start == 0, jnp.float32(0.0), g)
        plsc.addupdate_scatter(acc_v, [k], run_sum, mask=last)
        return carry

    lax.fori_loop(0, _EW // _L, edge_body, 0)
    pltpu.sync_copy(acc_v, out_hbm.at[wid])


@functools.lru_cache(maxsize=1)
def _make_sc_scatter():
    return pl.kernel(
        _sc_scatter_body,
        out_type=jax.ShapeDtypeStruct((_NW, _NPAD), jnp.float32),
        mesh=plsc.VectorSubcoreMesh(core_axis_name="c", subcore_axis_name="s"),
        compiler_params=pltpu.CompilerParams(needs_layout_passes=False),
        scratch_types=[
            pltpu.VMEM((_EW,), jnp.int32),
            pltpu.VMEM((_EW,), jnp.int32),
            pltpu.VMEM((_EW,), jnp.float32),
            pltpu.VMEM((_NPAD,), jnp.float32),
            pltpu.VMEM((_NPAD,), jnp.float32),
            pltpu.VMEM((32,), jnp.int32),
            pltpu.VMEM((_L,), jnp.float32),
        ],
    )


def kernel(node_attrs, edge_indices, edge_attrs, node_indices, edge_batch_indices,
           instruction_batch, distribution, node_prop_similarities, relation_similarity,
           weight_node_properties, weight_edge, weight_node_score, weight_relation_score):
    s2d = pl.pallas_call(
        _edge_body,
        grid=(_GE,),
        in_specs=[
            pl.BlockSpec((_TE // 128, 128), lambda i: (i, 0)),
            pl.BlockSpec((_TE, _H), lambda i: (i, 0)),
            pl.BlockSpec((_B, _H), lambda i: (0, 0)),
            pl.BlockSpec((_H, _H), lambda i: (0, 0)),
            pl.BlockSpec((_H, 1), lambda i: (0, 0)),
        ],
        out_specs=pl.BlockSpec((_TE // 128, 128), lambda i: (i, 0)),
        out_shape=jax.ShapeDtypeStruct((_ER, 128), jnp.float32),
    )(edge_batch_indices.reshape(_ER, 128), edge_attrs, instruction_batch,
      weight_edge, weight_relation_score.reshape(_H, 1))

    dist_pad = jnp.pad(distribution, (0, _NPAD - _N))
    parts = _make_sc_scatter()(edge_indices[0], edge_indices[1],
                               s2d.reshape(_E), dist_pad)

    seg2d = jnp.pad(node_indices, (0, _NPAD - _N),
                    constant_values=-1).reshape(_RD, 128)
    xs2d = pl.pallas_call(
        _node_body,
        grid=(_GN,),
        in_specs=[
            pl.BlockSpec((_TN // 128, 128), lambda i: (i, 0)),
            pl.BlockSpec((_TN, _P, _H), lambda i: (i, 0, 0)),
            pl.BlockSpec((_B, _H), lambda i: (0, 0)),
            pl.BlockSpec((_B, _P), lambda i: (0, 0)),
            pl.BlockSpec((_P, _H, _H), lambda i: (0, 0, 0)),
            pl.BlockSpec((_H, 1), lambda i: (0, 0)),
        ],
        out_specs=pl.BlockSpec((_TN // 128, 128), lambda i: (i, 0)),
        out_shape=jax.ShapeDtypeStruct((_RD, 128), jnp.float32),
    )(seg2d, node_attrs, instruction_batch, node_prop_similarities,
      weight_node_properties, weight_node_score.reshape(_H, 1))

    out2 = pl.pallas_call(
        _combine_body,
        out_shape=jax.ShapeDtypeStruct((_RD, 128), jnp.float32),
    )(parts.reshape(_NW, _RD, 128), xs2d, seg2d,
      relation_similarity.reshape(1, _B))
    return out2.reshape(_NPAD)[:_N]
